# baseline (speedup 1.0000x reference)
import jax
import jax.numpy as jnp
from jax import lax
from jax.experimental import pallas as pl
from jax.experimental.pallas import tpu as pltpu
from jax.experimental.pallas import tpu_sc as plsc

VOCAB = 100000
DIM = 64
B = 4096
W = 20

NC = 2
NS = 16
NW = NC * NS

G = 128
IPW = B // NW
OPW = (B * W) // NW
NG = OPW // G
GPC = 5
CR = G * GPC
NSC = NG // GPC
NSLICE = DIM // 16


def _body(ivectors, ovectors, iw, ow, nw, out_i, out_o, out_n,
          idx_i, idx_o, idx_n, buf0, buf1, sem0, sem1, semi):
    wid = lax.axis_index("s") * NC + lax.axis_index("c")

    pltpu.sync_copy(iw.at[pl.ds(pl.multiple_of(wid * IPW, IPW), IPW)], idx_i)
    pltpu.sync_copy(ow.at[pl.ds(pl.multiple_of(wid * OPW, OPW), OPW)], idx_o)
    pltpu.sync_copy(nw.at[pl.ds(pl.multiple_of(wid * OPW, OPW), OPW)], idx_n)

    bufs = (buf0, buf1)
    sems = (sem0, sem1)
    base_o = wid * OPW

    work = ([(idx_o, out_o, sc, False) for sc in range(NSC)]
            + [(idx_n, out_n, sc, True) for sc in range(NSC)])

    def issue(item, slot):
        idx, _, sc, _ = item
        return [
            pltpu.async_copy(
                ovectors.at[idx.at[pl.ds((sc * GPC + k) * G, G)]],
                bufs[slot].at[pl.ds(k * G, G)],
                sems[slot],
            )
            for k in range(GPC)
        ]

    handles = issue(work[0], 0)
    ih = pltpu.async_copy(ivectors.at[idx_i], buf1.at[pl.ds(0, G)], semi)

    ih.wait()
    pltpu.sync_copy(buf1.at[pl.ds(0, G)], out_i.at[pl.ds(wid * IPW, G)])

    for i, item in enumerate(work):
        cur = i % 2
        if i + 1 < len(work):
            nxt_handles = issue(work[i + 1], 1 - cur)
        for h in handles:
            h.wait()
        _, out, sc, negate = item
        buf = bufs[cur]
        if negate:
            def neg_rows(r, c):
                for rr in range(4):
                    for col in range(NSLICE):
                        s = pl.ds(col * 16, 16)
                        buf[r * 4 + rr, s] = -buf[r * 4 + rr, s]
                return c

            lax.fori_loop(0, CR // 4, neg_rows, 0)
        pltpu.sync_copy(buf, out.at[pl.ds(base_o + sc * CR, CR)])
        if i + 1 < len(work):
            handles = nxt_handles


@jax.jit
def kernel(iwords, owords, nwords, ivectors, ovectors):
    iw = iwords.astype(jnp.int32)
    ow = owords.astype(jnp.int32).reshape(-1)
    nw = nwords.astype(jnp.int32).reshape(-1)

    mesh = plsc.VectorSubcoreMesh(core_axis_name="c", subcore_axis_name="s")
    out_i, out_o, out_n = pl.kernel(
        _body,
        out_type=(
            jax.ShapeDtypeStruct((B, DIM), jnp.float32),
            jax.ShapeDtypeStruct((B * W, DIM), jnp.float32),
            jax.ShapeDtypeStruct((B * W, DIM), jnp.float32),
        ),
        mesh=mesh,
        compiler_params=pltpu.CompilerParams(use_tc_tiling_on_sc=False),
        scratch_types=[
            pltpu.VMEM((IPW,), jnp.int32),
            pltpu.VMEM((OPW,), jnp.int32),
            pltpu.VMEM((OPW,), jnp.int32),
            pltpu.VMEM((CR, DIM), jnp.float32),
            pltpu.VMEM((CR, DIM), jnp.float32),
            pltpu.SemaphoreType.DMA,
            pltpu.SemaphoreType.DMA,
            pltpu.SemaphoreType.DMA,
        ],
    )(ivectors, ovectors, iw, ow, nw)

    return (out_i,
            out_o.reshape(B, W, DIM),
            out_n.reshape(B, W, DIM))


# speedup vs baseline: 1.0455x; 1.0455x over previous
import jax
import jax.numpy as jnp
from jax import lax
from jax.experimental import pallas as pl
from jax.experimental.pallas import tpu as pltpu
from jax.experimental.pallas import tpu_sc as plsc

VOCAB = 100000
DIM = 64
B = 4096
W = 20

NC = 2
NS = 16
NW = NC * NS

G = 128
IPW = B // NW
OPW = (B * W) // NW
NG = OPW // G
GPC = 5
CR = G * GPC
NSC = NG // GPC
NSLICE = DIM // 16


def _body(ivectors, ovectors, iw, ow, nw, out_i, out_o, out_n,
          idx_i, idx_o, idx_n, buf0, buf1, sem0, sem1, semi):
    wid = lax.axis_index("s") * NC + lax.axis_index("c")

    pltpu.sync_copy(iw.at[pl.ds(pl.multiple_of(wid * IPW, IPW), IPW)], idx_i)
    pltpu.sync_copy(ow.at[pl.ds(pl.multiple_of(wid * OPW, OPW), OPW)], idx_o)
    pltpu.sync_copy(nw.at[pl.ds(pl.multiple_of(wid * OPW, OPW), OPW)], idx_n)

    bufs = (buf0, buf1)
    sems = (sem0, sem1)
    base_o = wid * OPW

    work = ([(idx_o, out_o, sc, False) for sc in range(NSC)]
            + [(idx_n, out_n, sc, True) for sc in range(NSC)])

    def issue(item, slot):
        idx, _, sc, _ = item
        return [
            pltpu.async_copy(
                ovectors.at[idx.at[pl.ds((sc * GPC + k) * G, G)]],
                bufs[slot].at[pl.ds(k * G, G)],
                sems[slot],
            )
            for k in range(GPC)
        ]

    handles = issue(work[0], 0)
    ih = pltpu.async_copy(ivectors.at[idx_i], buf1.at[pl.ds(0, G)], semi)

    ih.wait()
    pltpu.sync_copy(buf1.at[pl.ds(0, G)], out_i.at[pl.ds(wid * IPW, G)])

    for i, item in enumerate(work):
        cur = i % 2
        if i + 1 < len(work):
            nxt_handles = issue(work[i + 1], 1 - cur)
        for h in handles:
            h.wait()
        _, out, sc, negate = item
        buf = bufs[cur]
        if negate:
            def neg_rows(r, c):
                for rr in range(4):
                    for col in range(NSLICE):
                        s = pl.ds(col * 16, 16)
                        buf[r * 4 + rr, s] = -buf[r * 4 + rr, s]
                return c

            lax.fori_loop(0, CR // 4, neg_rows, 0)
        pltpu.sync_copy(buf, out.at[pl.ds(base_o + sc * CR, CR)])
        if i + 1 < len(work):
            handles = nxt_handles


@jax.jit
def kernel(iwords, owords, nwords, ivectors, ovectors):
    iw = iwords.astype(jnp.int32)
    ow = owords.astype(jnp.int32).T.reshape(-1)
    nw = nwords.astype(jnp.int32).T.reshape(-1)

    mesh = plsc.VectorSubcoreMesh(core_axis_name="c", subcore_axis_name="s")
    out_i, out_o, out_n = pl.kernel(
        _body,
        out_type=(
            jax.ShapeDtypeStruct((B, DIM), jnp.float32),
            jax.ShapeDtypeStruct((B * W, DIM), jnp.float32),
            jax.ShapeDtypeStruct((B * W, DIM), jnp.float32),
        ),
        mesh=mesh,
        compiler_params=pltpu.CompilerParams(use_tc_tiling_on_sc=False),
        scratch_types=[
            pltpu.VMEM((IPW,), jnp.int32),
            pltpu.VMEM((OPW,), jnp.int32),
            pltpu.VMEM((OPW,), jnp.int32),
            pltpu.VMEM((CR, DIM), jnp.float32),
            pltpu.VMEM((CR, DIM), jnp.float32),
            pltpu.SemaphoreType.DMA,
            pltpu.SemaphoreType.DMA,
            pltpu.SemaphoreType.DMA,
        ],
    )(ivectors, ovectors, iw, ow, nw)

    return (out_i,
            out_o.reshape(W, B, DIM).transpose(1, 0, 2),
            out_n.reshape(W, B, DIM).transpose(1, 0, 2))


# speedup vs baseline: 1.6753x; 1.6024x over previous
import jax
import jax.numpy as jnp
from jax import lax
from jax.experimental import pallas as pl
from jax.experimental.pallas import tpu as pltpu
from jax.experimental.pallas import tpu_sc as plsc

VOCAB = 100000
DIM = 64
B = 4096
W = 20
BW = B * W

NC = 2
NS = 16
NW = NC * NS
DPW = DIM // NW

UNROLL = 4
NIT = B // (16 * UNROLL)


def _body(tt_i, tt_o, iw, ow, nw, oi, oo, on,
          trow, idx0, idx1, out0, out1, shidx,
          si0, si1, so0, so1):
    cid = lax.axis_index("c")
    sid = lax.axis_index("s")

    @pl.when(sid == 0)
    def _stage():
        pltpu.sync_copy(ow, shidx.at[pl.ds(0, BW)])
        pltpu.sync_copy(nw, shidx.at[pl.ds(BW, BW)])
        pltpu.sync_copy(iw, shidx.at[pl.ds(2 * BW, B)])

    plsc.subcore_barrier()

    idx_bufs = (idx0, idx1)
    out_bufs = (out0, out1)
    idx_sems = (si0, si1)
    out_sems = (so0, so1)

    comp = []
    for j in range(DPW):
        comp += [("o", w, j) for w in range(W)]
        comp += [("n", w, j) for w in range(W)]
        comp += [("i", 0, j)]
    NP = len(comp)

    def idx_off(kind, w):
        if kind == "o":
            return w * B
        if kind == "n":
            return BW + w * B
        return 2 * BW

    def d_of(j):
        return (sid * NC + cid) * DPW + j

    idx_h = [None, None]
    out_h = [None, None]

    k0, w0, _ = comp[0]
    idx_h[0] = pltpu.async_copy(
        shidx.at[pl.ds(idx_off(k0, w0), B)], idx_bufs[0], idx_sems[0])

    for p, (kind, w, j) in enumerate(comp):
        slot = p % 2
        if kind == "o" and w == 0:
            pltpu.sync_copy(tt_o.at[d_of(j)], trow)
        elif kind == "i":
            pltpu.sync_copy(tt_i.at[d_of(j)], trow)

        if p + 1 < NP:
            kn, wn, _ = comp[p + 1]
            idx_h[1 - slot] = pltpu.async_copy(
                shidx.at[pl.ds(idx_off(kn, wn), B)],
                idx_bufs[1 - slot], idx_sems[1 - slot])

        idx_h[slot].wait()
        if out_h[slot] is not None:
            out_h[slot].wait()

        ib = idx_bufs[slot]
        ob = out_bufs[slot]
        neg = kind == "n"

        def gstep(i, c, ib=ib, ob=ob, neg=neg):
            base = i * (16 * UNROLL)
            for u in range(UNROLL):
                s = pl.ds(base + u * 16, 16)
                g = plsc.load_gather(trow, [ib[s]])
                ob[s] = -g if neg else g
            return c

        lax.fori_loop(0, NIT, gstep, 0)

        if kind == "i":
            dst = oi.at[d_of(j)]
        elif kind == "o":
            dst = oo.at[w, d_of(j)]
        else:
            dst = on.at[w, d_of(j)]
        out_h[slot] = pltpu.async_copy(ob, dst, out_sems[slot])

    out_h[0].wait()
    out_h[1].wait()


@jax.jit
def kernel(iwords, owords, nwords, ivectors, ovectors):
    tt_i = ivectors.T
    tt_o = ovectors.T
    iw = iwords.astype(jnp.int32)
    ow = owords.astype(jnp.int32).T.reshape(-1)
    nw = nwords.astype(jnp.int32).T.reshape(-1)

    mesh = plsc.VectorSubcoreMesh(core_axis_name="c", subcore_axis_name="s")
    oi, oo, on = pl.kernel(
        _body,
        out_type=(
            jax.ShapeDtypeStruct((DIM, B), jnp.float32),
            jax.ShapeDtypeStruct((W, DIM, B), jnp.float32),
            jax.ShapeDtypeStruct((W, DIM, B), jnp.float32),
        ),
        mesh=mesh,
        compiler_params=pltpu.CompilerParams(
            use_tc_tiling_on_sc=True, needs_layout_passes=False),
        scratch_types=[
            pltpu.VMEM((VOCAB,), jnp.float32),
            pltpu.VMEM((B,), jnp.int32),
            pltpu.VMEM((B,), jnp.int32),
            pltpu.VMEM((B,), jnp.float32),
            pltpu.VMEM((B,), jnp.float32),
            pltpu.VMEM_SHARED((2 * BW + B,), jnp.int32),
            pltpu.SemaphoreType.DMA,
            pltpu.SemaphoreType.DMA,
            pltpu.SemaphoreType.DMA,
            pltpu.SemaphoreType.DMA,
        ],
    )(tt_i, tt_o, iw, ow, nw)

    return (oi.T, oo.transpose(2, 0, 1), on.transpose(2, 0, 1))


# speedup vs baseline: 3.1780x; 1.8970x over previous
import jax
import jax.numpy as jnp
from jax import lax
from jax.experimental import pallas as pl
from jax.experimental.pallas import tpu as pltpu
from jax.experimental.pallas import tpu_sc as plsc

VOCAB = 100000
DIM = 64
B = 4096
W = 20
BW = B * W

NC = 2
NS = 16
NW = NC * NS
DPW = DIM // NW

UNROLL = 4
NIT = B // (16 * UNROLL)


def _body(tt_i, tt_o, iw, ow, nw, oi, oo, on,
          trow, idx0, idx1, out0, out1, shidx,
          si0, si1, so0, so1):
    cid = lax.axis_index("c")
    sid = lax.axis_index("s")

    @pl.when(sid == 0)
    def _stage():
        pltpu.sync_copy(ow, shidx.at[pl.ds(0, BW)])
        pltpu.sync_copy(nw, shidx.at[pl.ds(BW, BW)])
        pltpu.sync_copy(iw, shidx.at[pl.ds(2 * BW, B)])

    plsc.subcore_barrier()

    idx_bufs = (idx0, idx1)
    out_bufs = (out0, out1)
    idx_sems = (si0, si1)
    out_sems = (so0, so1)

    comp = []
    for j in range(DPW):
        comp += [("o", w, j) for w in range(W)]
        comp += [("n", w, j) for w in range(W)]
        comp += [("i", 0, j)]
    NP = len(comp)

    def idx_off(kind, w):
        if kind == "o":
            return w * B
        if kind == "n":
            return BW + w * B
        return 2 * BW

    def d_of(j):
        return (sid * NC + cid) * DPW + j

    idx_h = [None, None]
    out_h = [None, None]

    k0, w0, _ = comp[0]
    idx_h[0] = pltpu.async_copy(
        shidx.at[pl.ds(idx_off(k0, w0), B)], idx_bufs[0], idx_sems[0])

    for p, (kind, w, j) in enumerate(comp):
        slot = p % 2
        if kind == "o" and w == 0:
            pltpu.sync_copy(tt_o.at[d_of(j)], trow)
        elif kind == "i":
            pltpu.sync_copy(tt_i.at[d_of(j)], trow)

        if p + 1 < NP:
            kn, wn, _ = comp[p + 1]
            idx_h[1 - slot] = pltpu.async_copy(
                shidx.at[pl.ds(idx_off(kn, wn), B)],
                idx_bufs[1 - slot], idx_sems[1 - slot])

        idx_h[slot].wait()
        if out_h[slot] is not None:
            out_h[slot].wait()

        ib = idx_bufs[slot]
        ob = out_bufs[slot]
        neg = kind == "n"

        @plsc.parallel_loop(0, B // 16, step=1, unroll=UNROLL)
        def gstep(i, ib=ib, ob=ob, neg=neg):
            s = pl.ds(i * 16, 16)
            g = plsc.load_gather(trow, [ib[s]])
            ob[s] = -g if neg else g

        if kind == "i":
            dst = oi.at[d_of(j)]
        elif kind == "o":
            dst = oo.at[w, d_of(j)]
        else:
            dst = on.at[w, d_of(j)]
        out_h[slot] = pltpu.async_copy(ob, dst, out_sems[slot])

    out_h[0].wait()
    out_h[1].wait()


@jax.jit
def kernel(iwords, owords, nwords, ivectors, ovectors):
    tt_i = ivectors.T
    tt_o = ovectors.T
    iw = iwords.astype(jnp.int32)
    ow = owords.astype(jnp.int32).T.reshape(-1)
    nw = nwords.astype(jnp.int32).T.reshape(-1)

    mesh = plsc.VectorSubcoreMesh(core_axis_name="c", subcore_axis_name="s")
    oi, oo, on = pl.kernel(
        _body,
        out_type=(
            jax.ShapeDtypeStruct((DIM, B), jnp.float32),
            jax.ShapeDtypeStruct((W, DIM, B), jnp.float32),
            jax.ShapeDtypeStruct((W, DIM, B), jnp.float32),
        ),
        mesh=mesh,
        compiler_params=pltpu.CompilerParams(
            use_tc_tiling_on_sc=True, needs_layout_passes=False),
        scratch_types=[
            pltpu.VMEM((VOCAB,), jnp.float32),
            pltpu.VMEM((B,), jnp.int32),
            pltpu.VMEM((B,), jnp.int32),
            pltpu.VMEM((B,), jnp.float32),
            pltpu.VMEM((B,), jnp.float32),
            pltpu.VMEM_SHARED((2 * BW + B,), jnp.int32),
            pltpu.SemaphoreType.DMA,
            pltpu.SemaphoreType.DMA,
            pltpu.SemaphoreType.DMA,
            pltpu.SemaphoreType.DMA,
        ],
    )(tt_i, tt_o, iw, ow, nw)

    return (oi.T, oo.transpose(2, 0, 1), on.transpose(2, 0, 1))


# speedup vs baseline: 3.2265x; 1.0152x over previous
import jax
import jax.numpy as jnp
from jax import lax
from jax.experimental import pallas as pl
from jax.experimental.pallas import tpu as pltpu
from jax.experimental.pallas import tpu_sc as plsc

VOCAB = 100000
DIM = 64
B = 4096
W = 20
BW = B * W

NC = 2
NS = 16
NW = NC * NS
DPW = DIM // NW

UNROLL = 8
NIT = B // (16 * UNROLL)


def _body(tt_i, tt_o, iw, ow, nw, oi, oo, on,
          trow, idx0, idx1, out0, out1, shidx,
          si0, si1, so0, so1):
    cid = lax.axis_index("c")
    sid = lax.axis_index("s")

    @pl.when(sid == 0)
    def _stage():
        pltpu.sync_copy(ow, shidx.at[pl.ds(0, BW)])
        pltpu.sync_copy(nw, shidx.at[pl.ds(BW, BW)])
        pltpu.sync_copy(iw, shidx.at[pl.ds(2 * BW, B)])

    plsc.subcore_barrier()

    idx_bufs = (idx0, idx1)
    out_bufs = (out0, out1)
    idx_sems = (si0, si1)
    out_sems = (so0, so1)

    comp = []
    for j in range(DPW):
        comp += [("o", w, j) for w in range(W)]
        comp += [("n", w, j) for w in range(W)]
        comp += [("i", 0, j)]
    NP = len(comp)

    def idx_off(kind, w):
        if kind == "o":
            return w * B
        if kind == "n":
            return BW + w * B
        return 2 * BW

    def d_of(j):
        return (sid * NC + cid) * DPW + j

    idx_h = [None, None]
    out_h = [None, None]

    k0, w0, _ = comp[0]
    idx_h[0] = pltpu.async_copy(
        shidx.at[pl.ds(idx_off(k0, w0), B)], idx_bufs[0], idx_sems[0])

    for p, (kind, w, j) in enumerate(comp):
        slot = p % 2
        if kind == "o" and w == 0:
            pltpu.sync_copy(tt_o.at[d_of(j)], trow)
        elif kind == "i":
            pltpu.sync_copy(tt_i.at[d_of(j)], trow)

        if p + 1 < NP:
            kn, wn, _ = comp[p + 1]
            idx_h[1 - slot] = pltpu.async_copy(
                shidx.at[pl.ds(idx_off(kn, wn), B)],
                idx_bufs[1 - slot], idx_sems[1 - slot])

        idx_h[slot].wait()
        if out_h[slot] is not None:
            out_h[slot].wait()

        ib = idx_bufs[slot]
        ob = out_bufs[slot]
        neg = kind == "n"

        @plsc.parallel_loop(0, B // 16, step=1, unroll=UNROLL)
        def gstep(i, ib=ib, ob=ob, neg=neg):
            s = pl.ds(i * 16, 16)
            g = plsc.load_gather(trow, [ib[s]])
            ob[s] = -g if neg else g

        if kind == "i":
            dst = oi.at[d_of(j)]
        elif kind == "o":
            dst = oo.at[w, d_of(j)]
        else:
            dst = on.at[w, d_of(j)]
        out_h[slot] = pltpu.async_copy(ob, dst, out_sems[slot])

    out_h[0].wait()
    out_h[1].wait()


@jax.jit
def kernel(iwords, owords, nwords, ivectors, ovectors):
    tt_i = ivectors.T
    tt_o = ovectors.T
    iw = iwords.astype(jnp.int32)
    ow = owords.astype(jnp.int32).T.reshape(-1)
    nw = nwords.astype(jnp.int32).T.reshape(-1)

    mesh = plsc.VectorSubcoreMesh(core_axis_name="c", subcore_axis_name="s")
    oi, oo, on = pl.kernel(
        _body,
        out_type=(
            jax.ShapeDtypeStruct((DIM, B), jnp.float32),
            jax.ShapeDtypeStruct((W, DIM, B), jnp.float32),
            jax.ShapeDtypeStruct((W, DIM, B), jnp.float32),
        ),
        mesh=mesh,
        compiler_params=pltpu.CompilerParams(
            use_tc_tiling_on_sc=True, needs_layout_passes=False),
        scratch_types=[
            pltpu.VMEM((VOCAB,), jnp.float32),
            pltpu.VMEM((B,), jnp.int32),
            pltpu.VMEM((B,), jnp.int32),
            pltpu.VMEM((B,), jnp.float32),
            pltpu.VMEM((B,), jnp.float32),
            pltpu.VMEM_SHARED((2 * BW + B,), jnp.int32),
            pltpu.SemaphoreType.DMA,
            pltpu.SemaphoreType.DMA,
            pltpu.SemaphoreType.DMA,
            pltpu.SemaphoreType.DMA,
        ],
    )(tt_i, tt_o, iw, ow, nw)

    return (oi.T, oo.transpose(2, 0, 1), on.transpose(2, 0, 1))


# speedup vs baseline: 3.3316x; 1.0326x over previous
import jax
import jax.numpy as jnp
from jax import lax
from jax.experimental import pallas as pl
from jax.experimental.pallas import tpu as pltpu
from jax.experimental.pallas import tpu_sc as plsc

VOCAB = 100000
DIM = 64
B = 4096
W = 20
BW = B * W

NC = 2
NS = 16
NW = NC * NS
DPW = DIM // NW

UNROLL = 8
NIT = B // (16 * UNROLL)


def _body(tt_i, tt_o, iw, ow, nw, oi, oo, on,
          trow, idx0, idx1, out0, out1, shidx,
          si0, si1, so0, so1, st):
    cid = lax.axis_index("c")
    sid = lax.axis_index("s")

    def d_of(j):
        return (sid * NC + cid) * DPW + j

    trow_h = pltpu.async_copy(tt_o.at[d_of(0)], trow, st)

    QW = BW // 4
    for q in range(4):
        @pl.when(sid == q)
        def _stage_o(q=q):
            pltpu.sync_copy(ow.at[pl.ds(q * QW, QW)],
                            shidx.at[pl.ds(q * QW, QW)])

        @pl.when(sid == 4 + q)
        def _stage_n(q=q):
            pltpu.sync_copy(nw.at[pl.ds(q * QW, QW)],
                            shidx.at[pl.ds(BW + q * QW, QW)])

    @pl.when(sid == 8)
    def _stage_i():
        pltpu.sync_copy(iw, shidx.at[pl.ds(2 * BW, B)])

    plsc.subcore_barrier()
    trow_h.wait()

    idx_bufs = (idx0, idx1)
    out_bufs = (out0, out1)
    idx_sems = (si0, si1)
    out_sems = (so0, so1)

    comp = []
    for j in range(DPW):
        comp += [("o", w, j) for w in range(W)]
        comp += [("n", w, j) for w in range(W)]
        comp += [("i", 0, j)]
    NP = len(comp)

    def idx_off(kind, w):
        if kind == "o":
            return w * B
        if kind == "n":
            return BW + w * B
        return 2 * BW

    idx_h = [None, None]
    out_h = [None, None]

    k0, w0, _ = comp[0]
    idx_h[0] = pltpu.async_copy(
        shidx.at[pl.ds(idx_off(k0, w0), B)], idx_bufs[0], idx_sems[0])

    for p, (kind, w, j) in enumerate(comp):
        slot = p % 2
        if kind == "o" and w == 0 and p > 0:
            pltpu.sync_copy(tt_o.at[d_of(j)], trow)
        elif kind == "i":
            pltpu.sync_copy(tt_i.at[d_of(j)], trow)

        if p + 1 < NP:
            kn, wn, _ = comp[p + 1]
            idx_h[1 - slot] = pltpu.async_copy(
                shidx.at[pl.ds(idx_off(kn, wn), B)],
                idx_bufs[1 - slot], idx_sems[1 - slot])

        idx_h[slot].wait()
        if out_h[slot] is not None:
            out_h[slot].wait()

        ib = idx_bufs[slot]
        ob = out_bufs[slot]
        neg = kind == "n"

        @plsc.parallel_loop(0, B // 16, step=1, unroll=UNROLL)
        def gstep(i, ib=ib, ob=ob, neg=neg):
            s = pl.ds(i * 16, 16)
            g = plsc.load_gather(trow, [ib[s]])
            ob[s] = -g if neg else g

        if kind == "i":
            dst = oi.at[d_of(j)]
        elif kind == "o":
            dst = oo.at[w, d_of(j)]
        else:
            dst = on.at[w, d_of(j)]
        out_h[slot] = pltpu.async_copy(ob, dst, out_sems[slot])

    out_h[0].wait()
    out_h[1].wait()


@jax.jit
def kernel(iwords, owords, nwords, ivectors, ovectors):
    tt_i = ivectors.T
    tt_o = ovectors.T
    iw = iwords.astype(jnp.int32)
    ow = owords.astype(jnp.int32).T.reshape(-1)
    nw = nwords.astype(jnp.int32).T.reshape(-1)

    mesh = plsc.VectorSubcoreMesh(core_axis_name="c", subcore_axis_name="s")
    oi, oo, on = pl.kernel(
        _body,
        out_type=(
            jax.ShapeDtypeStruct((DIM, B), jnp.float32),
            jax.ShapeDtypeStruct((W, DIM, B), jnp.float32),
            jax.ShapeDtypeStruct((W, DIM, B), jnp.float32),
        ),
        mesh=mesh,
        compiler_params=pltpu.CompilerParams(
            use_tc_tiling_on_sc=True, needs_layout_passes=False),
        scratch_types=[
            pltpu.VMEM((VOCAB,), jnp.float32),
            pltpu.VMEM((B,), jnp.int32),
            pltpu.VMEM((B,), jnp.int32),
            pltpu.VMEM((B,), jnp.float32),
            pltpu.VMEM((B,), jnp.float32),
            pltpu.VMEM_SHARED((2 * BW + B,), jnp.int32),
            pltpu.SemaphoreType.DMA,
            pltpu.SemaphoreType.DMA,
            pltpu.SemaphoreType.DMA,
            pltpu.SemaphoreType.DMA,
            pltpu.SemaphoreType.DMA,
        ],
    )(tt_i, tt_o, iw, ow, nw)

    return (oi.T, oo.transpose(2, 0, 1), on.transpose(2, 0, 1))


# speedup vs baseline: 3.3326x; 1.0003x over previous
import jax
import jax.numpy as jnp
from jax import lax
from jax.experimental import pallas as pl
from jax.experimental.pallas import tpu as pltpu
from jax.experimental.pallas import tpu_sc as plsc

VOCAB = 100000
DIM = 64
B = 4096
W = 20
BW = B * W

NC = 2
NS = 16
NW = NC * NS
DPW = DIM // NW

UNROLL = 8


def _body(tt_i, tt_o, iw, ow, nw, oi, oo, on,
          trow, idx0, idx1, out0, out1, shidx,
          si0, si1, so0, so1, st):
    cid = lax.axis_index("c")
    sid = lax.axis_index("s")

    def d_of(j):
        return (sid * NC + cid) * DPW + j

    trow_h = pltpu.async_copy(tt_o.at[d_of(0)], trow, st)

    QW = BW // 4
    for q in range(4):
        @pl.when(sid == q)
        def _stage_o(q=q):
            pltpu.sync_copy(ow.at[pl.ds(q * QW, QW)],
                            shidx.at[pl.ds(q * QW, QW)])

        @pl.when(sid == 4 + q)
        def _stage_n(q=q):
            pltpu.sync_copy(nw.at[pl.ds(q * QW, QW)],
                            shidx.at[pl.ds(BW + q * QW, QW)])

    @pl.when(sid == 8)
    def _stage_i():
        pltpu.sync_copy(iw, shidx.at[pl.ds(2 * BW, B)])

    plsc.subcore_barrier()
    trow_h.wait()

    idx_bufs = (idx0, idx1)
    out_bufs = (out0, out1)
    idx_sems = (si0, si1)
    out_sems = (so0, so1)

    comp = []
    for j in range(DPW):
        comp += [("o", w, j) for w in range(W)]
        comp += [("n", w, j) for w in range(W)]
        comp += [("i", 0, j)]
    NP = len(comp)

    def idx_off(kind, w):
        if kind == "o":
            return w * B
        if kind == "n":
            return BW + w * B
        return 2 * BW

    idx_h = [None, None]
    out_h = [None, None]

    k0, w0, _ = comp[0]
    idx_h[0] = pltpu.async_copy(
        shidx.at[pl.ds(idx_off(k0, w0), B)], idx_bufs[0], idx_sems[0])

    for p, (kind, w, j) in enumerate(comp):
        slot = p % 2
        if kind == "o" and w == 0 and p > 0:
            pltpu.sync_copy(tt_o.at[d_of(j)], trow)
        elif kind == "i":
            pltpu.sync_copy(tt_i.at[d_of(j)], trow)

        if p + 1 < NP:
            kn, wn, _ = comp[p + 1]
            idx_h[1 - slot] = pltpu.async_copy(
                shidx.at[pl.ds(idx_off(kn, wn), B)],
                idx_bufs[1 - slot], idx_sems[1 - slot])

        idx_h[slot].wait()
        if out_h[slot] is not None:
            out_h[slot].wait()

        ib = idx_bufs[slot]
        ob = out_bufs[slot]
        neg = kind == "n"

        @plsc.parallel_loop(0, B // 16, step=1, unroll=UNROLL)
        def gstep(i, ib=ib, ob=ob, neg=neg):
            s = pl.ds(i * 16, 16)
            g = plsc.load_gather(trow, [ib[s]])
            ob[s] = -g if neg else g

        if kind == "i":
            dst = oi.at[d_of(j)]
        elif kind == "o":
            dst = oo.at[w, d_of(j)]
        else:
            dst = on.at[w, d_of(j)]
        out_h[slot] = pltpu.async_copy(ob, dst, out_sems[slot])

    out_h[0].wait()
    out_h[1].wait()


@jax.jit
def kernel(iwords, owords, nwords, ivectors, ovectors):
    tt_i = ivectors.T
    tt_o = ovectors.T
    iw = iwords.astype(jnp.int32)
    ow = owords.astype(jnp.int32).T.reshape(-1)
    nw = nwords.astype(jnp.int32).T.reshape(-1)

    mesh = plsc.VectorSubcoreMesh(core_axis_name="c", subcore_axis_name="s")
    oi, oo, on = pl.kernel(
        _body,
        out_type=(
            jax.ShapeDtypeStruct((DIM, B), jnp.float32),
            jax.ShapeDtypeStruct((W, DIM, B), jnp.float32),
            jax.ShapeDtypeStruct((W, DIM, B), jnp.float32),
        ),
        mesh=mesh,
        compiler_params=pltpu.CompilerParams(
            use_tc_tiling_on_sc=True, needs_layout_passes=False),
        scratch_types=[
            pltpu.VMEM((VOCAB,), jnp.float32),
            pltpu.VMEM((B,), jnp.int32),
            pltpu.VMEM((B,), jnp.int32),
            pltpu.VMEM((B,), jnp.float32),
            pltpu.VMEM((B,), jnp.float32),
            pltpu.VMEM_SHARED((2 * BW + B,), jnp.int32),
            pltpu.SemaphoreType.DMA,
            pltpu.SemaphoreType.DMA,
            pltpu.SemaphoreType.DMA,
            pltpu.SemaphoreType.DMA,
            pltpu.SemaphoreType.DMA,
        ],
    )(tt_i, tt_o, iw, ow, nw)

    return (oi.T, oo.transpose(2, 0, 1), on.transpose(2, 0, 1))
